# baseline (speedup 1.0000x reference)
import functools

import jax
import jax.numpy as jnp
from jax import lax
from jax.experimental import pallas as pl
from jax.experimental.pallas import tpu as pltpu
from jax.experimental.pallas import tpu_sc as plsc

_KEYS = (
    (1832780943, 270669613),
    (64467757, 2916123636),
    (2465931498, 255383827),
    (3134548294, 894150801),
)
_THRESH = (2516583, 1677722, 3355444, 838861)

_ROTS = ((13, 15, 26, 6), (17, 29, 16, 24))


def _threefry_bits(cnt_u32, k0, k1):
    ks0 = jnp.uint32(k0)
    ks1 = jnp.uint32(k1)
    ks2 = jnp.uint32((k0 ^ k1 ^ 0x1BD11BDA) & 0xFFFFFFFF)
    ks = (ks0, ks1, ks2)
    x0 = jnp.full_like(cnt_u32, ks0)
    x1 = cnt_u32 + ks1
    for i in range(5):
        rots = _ROTS[i % 2]
        for r in rots:
            x0 = x0 + x1
            x1 = (x1 << jnp.uint32(r)) | (x1 >> jnp.uint32(32 - r))
            x1 = x1 ^ x0
        x0 = x0 + ks[(i + 1) % 3]
        x1 = x1 + ks[(i + 2) % 3] + jnp.uint32(i + 1)
    return x0 ^ x1


def _masks_for(cnt_u32):
    out = []
    for (k0, k1), t in zip(_KEYS, _THRESH):
        bits = _threefry_bits(cnt_u32, k0, k1)
        mant = jnp.right_shift(bits, jnp.uint32(9)).astype(jnp.int32)
        out.append(mant < t)
    return out


def _attrs_kernel(S, L, uid_ref, ts_ref, et_ref,
                  muid_ref, mts_ref, met_ref,
                  m0_ref, m1_ref, m2_ref, m3_ref):
    b = pl.program_id(0)
    shape = (S, L)
    s_io = jax.lax.broadcasted_iota(jnp.int32, shape, 0)
    l_io = jax.lax.broadcasted_iota(jnp.int32, shape, 1)
    j = b * (S * L) + s_io * L + l_io
    m_uid, m_exe, m_ts, m_et = _masks_for(j.astype(jnp.uint32))

    blk = S * L
    uid = uid_ref[...].reshape(shape)
    ts = ts_ref[...].reshape(shape)
    et = et_ref[...].reshape(shape)
    muid_ref[...] = jnp.where(m_uid, 0, uid).reshape(blk)
    mts_ref[...] = jnp.where(m_ts, jnp.float32(0), ts).reshape(blk)
    met_ref[...] = jnp.where(m_et, 0, et).reshape(blk)
    m0_ref[...] = m_uid.reshape(blk)
    m1_ref[...] = m_exe.reshape(blk)
    m2_ref[...] = m_ts.reshape(blk)
    m3_ref[...] = m_et.reshape(blk)


_EXE_K0, _EXE_K1 = _KEYS[1]
_EXE_T = _THRESH[1]


def _sc_factor_vreg(row0):
    cnt = (row0 + lax.broadcasted_iota(jnp.int32, (16,), 0)).astype(jnp.uint32)
    bits = _threefry_bits(cnt, _EXE_K0, _EXE_K1)
    mant = jnp.right_shift(bits, jnp.uint32(9)).astype(jnp.int32)
    return jnp.where(mant < _EXE_T, jnp.float32(0), jnp.float32(1))


def _splat_lane(fvec, r):
    idx = jnp.full((16,), 0, jnp.int32) + r
    return lax.gather(
        fvec, idx[:, None],
        lax.GatherDimensionNumbers(
            offset_dims=(), collapsed_slice_dims=(0,),
            start_index_map=(0,)),
        (1,), mode=lax.GatherScatterMode.PROMISE_IN_BOUNDS)


def _make_sc_exe(n, d, n_workers):
    CH = 192
    full, tail = divmod(n, CH)
    base_c, extra = divmod(full, n_workers)
    assert tail % 16 == 0

    mesh = plsc.VectorSubcoreMesh(core_axis_name="c", subcore_axis_name="s")

    @functools.partial(
        pl.kernel, mesh=mesh,
        out_type=jax.ShapeDtypeStruct((n, d), jnp.float32),
        scratch_types=[
            pltpu.VMEM((CH, d), jnp.float32),
            pltpu.VMEM((CH, d), jnp.float32),
            pltpu.VMEM((CH, d), jnp.float32),
            pltpu.VMEM((CH, d), jnp.float32),
            pltpu.SemaphoreType.DMA,
            pltpu.SemaphoreType.DMA,
            pltpu.SemaphoreType.DMA,
            pltpu.SemaphoreType.DMA,
        ],
    )
    def sc_exe(exe_hbm, out_hbm, ib0, ib1, ob0, ob1, si0, si1, so0, so1):
        c = lax.axis_index("c")
        s = lax.axis_index("s")
        wid = s * jnp.int32(2) + c
        ibufs, obufs = (ib0, ib1), (ob0, ob1)
        isems, osems = (si0, si1), (so0, so1)
        nch_w = base_c + jnp.where(wid < extra, 1, 0)

        def row0_of(g):
            return pl.multiple_of((wid + g * n_workers) * CH, CH)

        def compute_rows(ib, ob, row0, nrows):
            def grp_body(v, cc):
                fvec = _sc_factor_vreg(row0 + 16 * v)

                @plsc.parallel_loop(0, 16, unroll=4)
                def _(r):
                    fac = _splat_lane(fvec, r)
                    row = 16 * v + r
                    for c8 in range(d // 16):
                        sl = pl.ds(c8 * 16, 16)
                        ob[row, sl] = ib[row, sl] * fac

                return cc

            lax.fori_loop(0, nrows // 16, grp_body, jnp.int32(0))

        @pl.when(nch_w > 0)
        def _():
            pltpu.async_copy(exe_hbm.at[pl.ds(row0_of(0), CH)], ib0, si0)

        def chunk_body(g, carry):
            for slot in (0, 1):
                @pl.when(g % 2 == slot)
                def _():
                    @pl.when(g + 1 < nch_w)
                    def _():
                        pltpu.async_copy(
                            exe_hbm.at[pl.ds(row0_of(g + 1), CH)],
                            ibufs[1 - slot], isems[1 - slot])
                    pltpu.make_async_copy(
                        exe_hbm.at[pl.ds(0, CH)], ibufs[slot],
                        isems[slot]).wait()

                    @pl.when(g >= 2)
                    def _():
                        pltpu.make_async_copy(
                            obufs[slot], out_hbm.at[pl.ds(0, CH)],
                            osems[slot]).wait()
                    compute_rows(ibufs[slot], obufs[slot], row0_of(g), CH)
                    pltpu.async_copy(
                        obufs[slot], out_hbm.at[pl.ds(row0_of(g), CH)],
                        osems[slot])
            return carry

        lax.fori_loop(0, nch_w, chunk_body, jnp.int32(0))

        for slot in (0, 1):
            @pl.when((nch_w > slot) & (((nch_w - 1 - slot) % 2) == 0))
            def _():
                pltpu.make_async_copy(
                    ob0, out_hbm.at[pl.ds(0, CH)], so0).wait()

            @pl.when((nch_w > slot) & (((nch_w - 1 - slot) % 2) == 1))
            def _():
                pltpu.make_async_copy(
                    ob1, out_hbm.at[pl.ds(0, CH)], so1).wait()

        if tail:
            @pl.when(wid == 0)
            def _():
                row0 = full * CH
                pltpu.sync_copy(exe_hbm.at[pl.ds(row0, tail)],
                                ib0.at[pl.ds(0, tail)])
                compute_rows(ib0, ob0, row0, tail)
                pltpu.sync_copy(ob0.at[pl.ds(0, tail)],
                                out_hbm.at[pl.ds(row0, tail)])

    return sc_exe


def kernel(uid, exe_path, timestamp, edge_type):
    n, d = exe_path.shape
    S, L = 8, 1024
    blk = S * L
    G = -(-n // blk)

    blk1 = pl.BlockSpec((blk,), lambda b: (b,))
    stage_a = pl.pallas_call(
        functools.partial(_attrs_kernel, S, L),
        grid=(G,),
        in_specs=[blk1, blk1, blk1],
        out_specs=[blk1] * 7,
        out_shape=[
            jax.ShapeDtypeStruct((n,), uid.dtype),
            jax.ShapeDtypeStruct((n,), timestamp.dtype),
            jax.ShapeDtypeStruct((n,), edge_type.dtype),
            jax.ShapeDtypeStruct((n,), jnp.bool_),
            jax.ShapeDtypeStruct((n,), jnp.bool_),
            jax.ShapeDtypeStruct((n,), jnp.bool_),
            jax.ShapeDtypeStruct((n,), jnp.bool_),
        ],
        compiler_params=pltpu.CompilerParams(
            dimension_semantics=("arbitrary",)),
    )
    if True:
        muid = jnp.zeros((n,), uid.dtype)
        mts = jnp.zeros((n,), timestamp.dtype)
        met = jnp.zeros((n,), edge_type.dtype)
        m_uid = m_exe = m_ts = m_et = jnp.zeros((n,), jnp.bool_)
    else:
        (muid, mts, met, m_uid, m_exe, m_ts, m_et) = stage_a(
            uid, timestamp, edge_type)

    mexe = _make_sc_exe(n, d, 32)(exe_path)

    return muid, mexe, mts, met, m_uid, m_exe, m_ts, m_et


# speedup vs baseline: 1.0085x; 1.0085x over previous
import functools

import jax
import jax.numpy as jnp
from jax import lax
from jax.experimental import pallas as pl
from jax.experimental.pallas import tpu as pltpu
from jax.experimental.pallas import tpu_sc as plsc

_KEYS = (
    (1832780943, 270669613),
    (64467757, 2916123636),
    (2465931498, 255383827),
    (3134548294, 894150801),
)
_THRESH = (2516583, 1677722, 3355444, 838861)

_ROTS = ((13, 15, 26, 6), (17, 29, 16, 24))


def _threefry_bits(cnt_u32, k0, k1):
    ks0 = jnp.uint32(k0)
    ks1 = jnp.uint32(k1)
    ks2 = jnp.uint32((k0 ^ k1 ^ 0x1BD11BDA) & 0xFFFFFFFF)
    ks = (ks0, ks1, ks2)
    x0 = jnp.full_like(cnt_u32, ks0)
    x1 = cnt_u32 + ks1
    for i in range(5):
        rots = _ROTS[i % 2]
        for r in rots:
            x0 = x0 + x1
            x1 = (x1 << jnp.uint32(r)) | (x1 >> jnp.uint32(32 - r))
            x1 = x1 ^ x0
        x0 = x0 + ks[(i + 1) % 3]
        x1 = x1 + ks[(i + 2) % 3] + jnp.uint32(i + 1)
    return x0 ^ x1


def _masks_for(cnt_u32):
    out = []
    for (k0, k1), t in zip(_KEYS, _THRESH):
        bits = _threefry_bits(cnt_u32, k0, k1)
        mant = jnp.right_shift(bits, jnp.uint32(9)).astype(jnp.int32)
        out.append(mant < t)
    return out


def _attrs_kernel(S, L, uid_ref, ts_ref, et_ref,
                  muid_ref, mts_ref, met_ref,
                  m0_ref, m1_ref, m2_ref, m3_ref):
    b = pl.program_id(0)
    shape = (S, L)
    s_io = jax.lax.broadcasted_iota(jnp.int32, shape, 0)
    l_io = jax.lax.broadcasted_iota(jnp.int32, shape, 1)
    j = b * (S * L) + s_io * L + l_io
    m_uid, m_exe, m_ts, m_et = _masks_for(j.astype(jnp.uint32))

    blk = S * L
    uid = uid_ref[...].reshape(shape)
    ts = ts_ref[...].reshape(shape)
    et = et_ref[...].reshape(shape)
    muid_ref[...] = jnp.where(m_uid, 0, uid).reshape(blk)
    mts_ref[...] = jnp.where(m_ts, jnp.float32(0), ts).reshape(blk)
    met_ref[...] = jnp.where(m_et, 0, et).reshape(blk)
    m0_ref[...] = m_uid.reshape(blk)
    m1_ref[...] = m_exe.reshape(blk)
    m2_ref[...] = m_ts.reshape(blk)
    m3_ref[...] = m_et.reshape(blk)


_EXE_K0, _EXE_K1 = _KEYS[1]
_EXE_T = _THRESH[1]


def _sc_factor_vreg(row0):
    cnt = (row0 + lax.broadcasted_iota(jnp.int32, (16,), 0)).astype(jnp.uint32)
    bits = _threefry_bits(cnt, _EXE_K0, _EXE_K1)
    mant = jnp.right_shift(bits, jnp.uint32(9)).astype(jnp.int32)
    return jnp.where(mant < _EXE_T, jnp.float32(0), jnp.float32(1))


def _splat_lane(fvec, r):
    idx = jnp.full((16,), 0, jnp.int32) + r
    return lax.gather(
        fvec, idx[:, None],
        lax.GatherDimensionNumbers(
            offset_dims=(), collapsed_slice_dims=(0,),
            start_index_map=(0,)),
        (1,), mode=lax.GatherScatterMode.PROMISE_IN_BOUNDS)


def _make_sc_exe(n, d, n_workers, heavy_parity, heavy_cnt):
    CH = 192
    full, tail = divmod(n, CH)
    nh = n_workers // 2
    light_cnt, rem = divmod(full - nh * heavy_cnt, nh)
    assert 0 <= rem < nh and light_cnt > 0
    assert tail % 16 == 0

    mesh = plsc.VectorSubcoreMesh(core_axis_name="c", subcore_axis_name="s")

    @functools.partial(
        pl.kernel, mesh=mesh,
        out_type=jax.ShapeDtypeStruct((n, d), jnp.float32),
        scratch_types=[
            pltpu.VMEM((CH, d), jnp.float32),
            pltpu.VMEM((CH, d), jnp.float32),
            pltpu.VMEM((CH, d), jnp.float32),
            pltpu.VMEM((CH, d), jnp.float32),
            pltpu.SemaphoreType.DMA,
            pltpu.SemaphoreType.DMA,
            pltpu.SemaphoreType.DMA,
            pltpu.SemaphoreType.DMA,
        ],
    )
    def sc_exe(exe_hbm, out_hbm, ib0, ib1, ob0, ob1, si0, si1, so0, so1):
        c = lax.axis_index("c")
        s = lax.axis_index("s")
        wid = s * jnp.int32(2) + c
        ibufs, obufs = (ib0, ib1), (ob0, ob1)
        isems, osems = (si0, si1), (so0, so1)

        is_heavy = (wid % 2) == heavy_parity
        if heavy_parity == 0:
            h_before = (wid + 1) // 2
            l_before = wid // 2
        else:
            h_before = wid // 2
            l_before = (wid + 1) // 2
        e_before = jnp.minimum(h_before, rem)
        start_w = heavy_cnt * h_before + e_before + light_cnt * l_before
        hidx = h_before
        nch_w = jnp.where(is_heavy,
                          heavy_cnt + jnp.where(hidx < rem, 1, 0),
                          light_cnt)

        def row0_of(g):
            return pl.multiple_of((start_w + g) * CH, CH)

        def compute_rows(ib, ob, row0, nrows):
            def grp_body(v, cc):
                fvec = _sc_factor_vreg(row0 + 16 * v)

                @plsc.parallel_loop(0, 16, unroll=4)
                def _(r):
                    fac = _splat_lane(fvec, r)
                    row = 16 * v + r
                    for c8 in range(d // 16):
                        sl = pl.ds(c8 * 16, 16)
                        ob[row, sl] = ib[row, sl] * fac

                return cc

            lax.fori_loop(0, nrows // 16, grp_body, jnp.int32(0))

        @pl.when(nch_w > 0)
        def _():
            pltpu.async_copy(exe_hbm.at[pl.ds(row0_of(0), CH)], ib0, si0)

        def chunk_body(g, carry):
            for slot in (0, 1):
                @pl.when(g % 2 == slot)
                def _():
                    @pl.when(g + 1 < nch_w)
                    def _():
                        pltpu.async_copy(
                            exe_hbm.at[pl.ds(row0_of(g + 1), CH)],
                            ibufs[1 - slot], isems[1 - slot])
                    pltpu.make_async_copy(
                        exe_hbm.at[pl.ds(0, CH)], ibufs[slot],
                        isems[slot]).wait()

                    @pl.when(g >= 2)
                    def _():
                        pltpu.make_async_copy(
                            obufs[slot], out_hbm.at[pl.ds(0, CH)],
                            osems[slot]).wait()
                    compute_rows(ibufs[slot], obufs[slot], row0_of(g), CH)
                    pltpu.async_copy(
                        obufs[slot], out_hbm.at[pl.ds(row0_of(g), CH)],
                        osems[slot])
            return carry

        lax.fori_loop(0, nch_w, chunk_body, jnp.int32(0))

        for slot in (0, 1):
            @pl.when((nch_w > slot) & (((nch_w - 1 - slot) % 2) == 0))
            def _():
                pltpu.make_async_copy(
                    ob0, out_hbm.at[pl.ds(0, CH)], so0).wait()

            @pl.when((nch_w > slot) & (((nch_w - 1 - slot) % 2) == 1))
            def _():
                pltpu.make_async_copy(
                    ob1, out_hbm.at[pl.ds(0, CH)], so1).wait()

        if tail:
            tail_wid = n_workers - 1 if heavy_parity == 0 else n_workers - 2

            @pl.when(wid == tail_wid)
            def _():
                row0 = full * CH
                pltpu.sync_copy(exe_hbm.at[pl.ds(row0, tail)],
                                ib0.at[pl.ds(0, tail)])
                compute_rows(ib0, ob0, row0, tail)
                pltpu.sync_copy(ob0.at[pl.ds(0, tail)],
                                out_hbm.at[pl.ds(row0, tail)])

    return sc_exe


def kernel(uid, exe_path, timestamp, edge_type):
    n, d = exe_path.shape
    S, L = 8, 1024
    blk = S * L
    G = -(-n // blk)

    blk1 = pl.BlockSpec((blk,), lambda b: (b,))
    stage_a = pl.pallas_call(
        functools.partial(_attrs_kernel, S, L),
        grid=(G,),
        in_specs=[blk1, blk1, blk1],
        out_specs=[blk1] * 7,
        out_shape=[
            jax.ShapeDtypeStruct((n,), uid.dtype),
            jax.ShapeDtypeStruct((n,), timestamp.dtype),
            jax.ShapeDtypeStruct((n,), edge_type.dtype),
            jax.ShapeDtypeStruct((n,), jnp.bool_),
            jax.ShapeDtypeStruct((n,), jnp.bool_),
            jax.ShapeDtypeStruct((n,), jnp.bool_),
            jax.ShapeDtypeStruct((n,), jnp.bool_),
        ],
        compiler_params=pltpu.CompilerParams(
            dimension_semantics=("arbitrary",)),
    )
    (muid, mts, met, m_uid, m_exe, m_ts, m_et) = stage_a(
        uid, timestamp, edge_type)

    mexe = _make_sc_exe(n, d, 32, heavy_parity=0, heavy_cnt=20)(exe_path)

    return muid, mexe, mts, met, m_uid, m_exe, m_ts, m_et


# speedup vs baseline: 1.0266x; 1.0180x over previous
import functools

import jax
import jax.numpy as jnp
from jax import lax
from jax.experimental import pallas as pl
from jax.experimental.pallas import tpu as pltpu
from jax.experimental.pallas import tpu_sc as plsc

_KEYS = (
    (1832780943, 270669613),
    (64467757, 2916123636),
    (2465931498, 255383827),
    (3134548294, 894150801),
)
_THRESH = (2516583, 1677722, 3355444, 838861)

_ROTS = ((13, 15, 26, 6), (17, 29, 16, 24))


def _threefry_bits(cnt_u32, k0, k1):
    ks0 = jnp.uint32(k0)
    ks1 = jnp.uint32(k1)
    ks2 = jnp.uint32((k0 ^ k1 ^ 0x1BD11BDA) & 0xFFFFFFFF)
    ks = (ks0, ks1, ks2)
    x0 = jnp.full_like(cnt_u32, ks0)
    x1 = cnt_u32 + ks1
    for i in range(5):
        rots = _ROTS[i % 2]
        for r in rots:
            x0 = x0 + x1
            x1 = (x1 << jnp.uint32(r)) | (x1 >> jnp.uint32(32 - r))
            x1 = x1 ^ x0
        x0 = x0 + ks[(i + 1) % 3]
        x1 = x1 + ks[(i + 2) % 3] + jnp.uint32(i + 1)
    return x0 ^ x1


def _masks_for(cnt_u32):
    out = []
    for (k0, k1), t in zip(_KEYS, _THRESH):
        bits = _threefry_bits(cnt_u32, k0, k1)
        mant = jnp.right_shift(bits, jnp.uint32(9)).astype(jnp.int32)
        out.append(mant < t)
    return out


def _attrs_kernel(S, L, uid_ref, ts_ref, et_ref,
                  muid_ref, mts_ref, met_ref,
                  m0_ref, m1_ref, m2_ref, m3_ref):
    b = pl.program_id(0)
    shape = (S, L)
    s_io = jax.lax.broadcasted_iota(jnp.int32, shape, 0)
    l_io = jax.lax.broadcasted_iota(jnp.int32, shape, 1)
    j = b * (S * L) + s_io * L + l_io
    m_uid, m_exe, m_ts, m_et = _masks_for(j.astype(jnp.uint32))

    blk = S * L
    uid = uid_ref[...].reshape(shape)
    ts = ts_ref[...].reshape(shape)
    et = et_ref[...].reshape(shape)
    muid_ref[...] = jnp.where(m_uid, 0, uid).reshape(blk)
    mts_ref[...] = jnp.where(m_ts, jnp.float32(0), ts).reshape(blk)
    met_ref[...] = jnp.where(m_et, 0, et).reshape(blk)
    m0_ref[...] = m_uid.reshape(blk)
    m1_ref[...] = m_exe.reshape(blk)
    m2_ref[...] = m_ts.reshape(blk)
    m3_ref[...] = m_et.reshape(blk)


_EXE_K0, _EXE_K1 = _KEYS[1]
_EXE_T = _THRESH[1]


def _sc_factor_vreg(row0):
    cnt = (row0 + lax.broadcasted_iota(jnp.int32, (16,), 0)).astype(jnp.uint32)
    bits = _threefry_bits(cnt, _EXE_K0, _EXE_K1)
    mant = jnp.right_shift(bits, jnp.uint32(9)).astype(jnp.int32)
    return jnp.where(mant < _EXE_T, jnp.float32(0), jnp.float32(1))


def _splat_lane(fvec, r):
    idx = jnp.full((16,), 0, jnp.int32) + r
    return lax.gather(
        fvec, idx[:, None],
        lax.GatherDimensionNumbers(
            offset_dims=(), collapsed_slice_dims=(0,),
            start_index_map=(0,)),
        (1,), mode=lax.GatherScatterMode.PROMISE_IN_BOUNDS)


def _make_sc_exe(n, d, n_workers, heavy_parity, heavy_cnt):
    CH = 192
    full, tail = divmod(n, CH)
    nh = n_workers // 2
    light_cnt, rem = divmod(full - nh * heavy_cnt, nh)
    assert 0 <= rem < nh and light_cnt > 0
    assert tail % 16 == 0

    mesh = plsc.VectorSubcoreMesh(core_axis_name="c", subcore_axis_name="s")

    @functools.partial(
        pl.kernel, mesh=mesh,
        out_type=jax.ShapeDtypeStruct((n, d), jnp.float32),
        scratch_types=[
            pltpu.VMEM((CH, d), jnp.float32),
            pltpu.VMEM((CH, d), jnp.float32),
            pltpu.VMEM((CH, d), jnp.float32),
            pltpu.VMEM((CH, d), jnp.float32),
            pltpu.SemaphoreType.DMA,
            pltpu.SemaphoreType.DMA,
            pltpu.SemaphoreType.DMA,
            pltpu.SemaphoreType.DMA,
        ],
    )
    def sc_exe(exe_hbm, out_hbm, ib0, ib1, ob0, ob1, si0, si1, so0, so1):
        c = lax.axis_index("c")
        s = lax.axis_index("s")
        wid = s * jnp.int32(2) + c
        ibufs, obufs = (ib0, ib1), (ob0, ob1)
        isems, osems = (si0, si1), (so0, so1)

        is_heavy = (wid % 2) == heavy_parity
        if heavy_parity == 0:
            h_before = (wid + 1) // 2
            l_before = wid // 2
        else:
            h_before = wid // 2
            l_before = (wid + 1) // 2
        e_before = jnp.minimum(h_before, rem)
        start_w = heavy_cnt * h_before + e_before + light_cnt * l_before
        hidx = h_before
        nch_w = jnp.where(is_heavy,
                          heavy_cnt + jnp.where(hidx < rem, 1, 0),
                          light_cnt)

        def row0_of(g):
            return pl.multiple_of((start_w + g) * CH, CH)

        def compute_rows(ib, ob, row0, nrows):
            def grp_body(v, cc):
                fvec = _sc_factor_vreg(row0 + 16 * v)

                @plsc.parallel_loop(0, 16, unroll=4)
                def _(r):
                    fac = _splat_lane(fvec, r)
                    row = 16 * v + r
                    for c8 in range(d // 16):
                        sl = pl.ds(c8 * 16, 16)
                        ob[row, sl] = ib[row, sl] * fac

                return cc

            lax.fori_loop(0, nrows // 16, grp_body, jnp.int32(0))

        @pl.when(nch_w > 0)
        def _():
            pltpu.async_copy(exe_hbm.at[pl.ds(row0_of(0), CH)], ib0, si0)

        def chunk_body(g, carry):
            for slot in (0, 1):
                @pl.when(g % 2 == slot)
                def _():
                    @pl.when(g + 1 < nch_w)
                    def _():
                        pltpu.async_copy(
                            exe_hbm.at[pl.ds(row0_of(g + 1), CH)],
                            ibufs[1 - slot], isems[1 - slot])
                    pltpu.make_async_copy(
                        exe_hbm.at[pl.ds(0, CH)], ibufs[slot],
                        isems[slot]).wait()

                    @pl.when(g >= 2)
                    def _():
                        pltpu.make_async_copy(
                            obufs[slot], out_hbm.at[pl.ds(0, CH)],
                            osems[slot]).wait()
                    compute_rows(ibufs[slot], obufs[slot], row0_of(g), CH)
                    pltpu.async_copy(
                        obufs[slot], out_hbm.at[pl.ds(row0_of(g), CH)],
                        osems[slot])
            return carry

        lax.fori_loop(0, nch_w, chunk_body, jnp.int32(0))

        for slot in (0, 1):
            @pl.when((nch_w > slot) & (((nch_w - 1 - slot) % 2) == 0))
            def _():
                pltpu.make_async_copy(
                    ob0, out_hbm.at[pl.ds(0, CH)], so0).wait()

            @pl.when((nch_w > slot) & (((nch_w - 1 - slot) % 2) == 1))
            def _():
                pltpu.make_async_copy(
                    ob1, out_hbm.at[pl.ds(0, CH)], so1).wait()

        if tail:
            tail_wid = n_workers - 1 if heavy_parity == 0 else n_workers - 2

            @pl.when(wid == tail_wid)
            def _():
                row0 = full * CH
                pltpu.sync_copy(exe_hbm.at[pl.ds(row0, tail)],
                                ib0.at[pl.ds(0, tail)])
                compute_rows(ib0, ob0, row0, tail)
                pltpu.sync_copy(ob0.at[pl.ds(0, tail)],
                                out_hbm.at[pl.ds(row0, tail)])

    return sc_exe


def kernel(uid, exe_path, timestamp, edge_type):
    n, d = exe_path.shape
    S, L = 8, 1024
    blk = S * L
    G = -(-n // blk)

    blk1 = pl.BlockSpec((blk,), lambda b: (b,))
    stage_a = pl.pallas_call(
        functools.partial(_attrs_kernel, S, L),
        grid=(G,),
        in_specs=[blk1, blk1, blk1],
        out_specs=[blk1] * 7,
        out_shape=[
            jax.ShapeDtypeStruct((n,), uid.dtype),
            jax.ShapeDtypeStruct((n,), timestamp.dtype),
            jax.ShapeDtypeStruct((n,), edge_type.dtype),
            jax.ShapeDtypeStruct((n,), jnp.bool_),
            jax.ShapeDtypeStruct((n,), jnp.bool_),
            jax.ShapeDtypeStruct((n,), jnp.bool_),
            jax.ShapeDtypeStruct((n,), jnp.bool_),
        ],
        compiler_params=pltpu.CompilerParams(
            dimension_semantics=("arbitrary",)),
    )
    (muid, mts, met, m_uid, m_exe, m_ts, m_et) = stage_a(
        uid, timestamp, edge_type)

    mexe = _make_sc_exe(n, d, 32, heavy_parity=1, heavy_cnt=20)(exe_path)

    return muid, mexe, mts, met, m_uid, m_exe, m_ts, m_et


# speedup vs baseline: 1.0666x; 1.0390x over previous
import functools

import jax
import jax.numpy as jnp
from jax import lax
from jax.experimental import pallas as pl
from jax.experimental.pallas import tpu as pltpu
from jax.experimental.pallas import tpu_sc as plsc

_KEYS = (
    (1832780943, 270669613),
    (64467757, 2916123636),
    (2465931498, 255383827),
    (3134548294, 894150801),
)
_THRESH = (2516583, 1677722, 3355444, 838861)

_ROTS = ((13, 15, 26, 6), (17, 29, 16, 24))


def _threefry_bits(cnt_u32, k0, k1):
    ks0 = jnp.uint32(k0)
    ks1 = jnp.uint32(k1)
    ks2 = jnp.uint32((k0 ^ k1 ^ 0x1BD11BDA) & 0xFFFFFFFF)
    ks = (ks0, ks1, ks2)
    x0 = jnp.full_like(cnt_u32, ks0)
    x1 = cnt_u32 + ks1
    for i in range(5):
        rots = _ROTS[i % 2]
        for r in rots:
            x0 = x0 + x1
            x1 = (x1 << jnp.uint32(r)) | (x1 >> jnp.uint32(32 - r))
            x1 = x1 ^ x0
        x0 = x0 + ks[(i + 1) % 3]
        x1 = x1 + ks[(i + 2) % 3] + jnp.uint32(i + 1)
    return x0 ^ x1


def _masks_for(cnt_u32):
    out = []
    for (k0, k1), t in zip(_KEYS, _THRESH):
        bits = _threefry_bits(cnt_u32, k0, k1)
        mant = jnp.right_shift(bits, jnp.uint32(9)).astype(jnp.int32)
        out.append(mant < t)
    return out


def _attrs_kernel(S, L, uid_ref, ts_ref, et_ref,
                  muid_ref, mts_ref, met_ref,
                  m0_ref, m1_ref, m2_ref, m3_ref):
    b = pl.program_id(0)
    shape = (S, L)
    s_io = jax.lax.broadcasted_iota(jnp.int32, shape, 0)
    l_io = jax.lax.broadcasted_iota(jnp.int32, shape, 1)
    j = b * (S * L) + s_io * L + l_io
    m_uid, m_exe, m_ts, m_et = _masks_for(j.astype(jnp.uint32))

    blk = S * L
    uid = uid_ref[...].reshape(shape)
    ts = ts_ref[...].reshape(shape)
    et = et_ref[...].reshape(shape)
    muid_ref[...] = jnp.where(m_uid, 0, uid).reshape(blk)
    mts_ref[...] = jnp.where(m_ts, jnp.float32(0), ts).reshape(blk)
    met_ref[...] = jnp.where(m_et, 0, et).reshape(blk)
    m0_ref[...] = m_uid.reshape(blk)
    m1_ref[...] = m_exe.reshape(blk)
    m2_ref[...] = m_ts.reshape(blk)
    m3_ref[...] = m_et.reshape(blk)


_EXE_K0, _EXE_K1 = _KEYS[1]
_EXE_T = _THRESH[1]


def _sc_factor_vreg(row0):
    cnt = (row0 + lax.broadcasted_iota(jnp.int32, (16,), 0)).astype(jnp.uint32)
    bits = _threefry_bits(cnt, _EXE_K0, _EXE_K1)
    mant = jnp.right_shift(bits, jnp.uint32(9)).astype(jnp.int32)
    return jnp.where(mant < _EXE_T, jnp.float32(0), jnp.float32(1))


def _splat_lane(fvec, r):
    idx = jnp.full((16,), 0, jnp.int32) + r
    return lax.gather(
        fvec, idx[:, None],
        lax.GatherDimensionNumbers(
            offset_dims=(), collapsed_slice_dims=(0,),
            start_index_map=(0,)),
        (1,), mode=lax.GatherScatterMode.PROMISE_IN_BOUNDS)


def _make_sc_exe(n, d, n_workers):
    CH = 192
    full, tail = divmod(n, CH)
    base_c, extra = divmod(full, n_workers)
    assert tail % 16 == 0

    mesh = plsc.VectorSubcoreMesh(core_axis_name="c", subcore_axis_name="s")

    @functools.partial(
        pl.kernel, mesh=mesh,
        out_type=jax.ShapeDtypeStruct((n, d), jnp.float32),
        scratch_types=[
            pltpu.VMEM((CH, d), jnp.float32),
            pltpu.VMEM((CH, d), jnp.float32),
            pltpu.VMEM((CH, d), jnp.float32),
            pltpu.VMEM((CH, d), jnp.float32),
            pltpu.SemaphoreType.DMA,
            pltpu.SemaphoreType.DMA,
            pltpu.SemaphoreType.DMA,
            pltpu.SemaphoreType.DMA,
        ],
    )
    def sc_exe(exe_hbm, out_hbm, ib0, ib1, ob0, ob1, si0, si1, so0, so1):
        c = lax.axis_index("c")
        s = lax.axis_index("s")
        wid = s * jnp.int32(2) + c
        ibufs, obufs = (ib0, ib1), (ob0, ob1)
        isems, osems = (si0, si1), (so0, so1)
        nch_w = base_c + jnp.where(wid < extra, 1, 0)

        def row0_of(g):
            return pl.multiple_of((wid + g * n_workers) * CH, CH)

        def compute_rows(ib, ob, row0, nrows):
            def grp_body(v, cc):
                fvec = _sc_factor_vreg(row0 + 16 * v)

                @plsc.parallel_loop(0, 16, unroll=4)
                def _(r):
                    fac = _splat_lane(fvec, r)
                    row = 16 * v + r
                    for c8 in range(d // 16):
                        sl = pl.ds(c8 * 16, 16)
                        ob[row, sl] = ib[row, sl] * fac

                return cc

            lax.fori_loop(0, nrows // 16, grp_body, jnp.int32(0))

        @pl.when(nch_w > 0)
        def _():
            pltpu.async_copy(exe_hbm.at[pl.ds(row0_of(0), CH)], ib0, si0)

        def chunk_body(g, carry):
            for slot in (0, 1):
                @pl.when(g % 2 == slot)
                def _():
                    @pl.when(g + 1 < nch_w)
                    def _():
                        pltpu.async_copy(
                            exe_hbm.at[pl.ds(row0_of(g + 1), CH)],
                            ibufs[1 - slot], isems[1 - slot])
                    pltpu.make_async_copy(
                        exe_hbm.at[pl.ds(0, CH)], ibufs[slot],
                        isems[slot]).wait()

                    @pl.when(g >= 2)
                    def _():
                        pltpu.make_async_copy(
                            obufs[slot], out_hbm.at[pl.ds(0, CH)],
                            osems[slot]).wait()
                    compute_rows(ibufs[slot], obufs[slot], row0_of(g), CH)
                    pltpu.async_copy(
                        obufs[slot], out_hbm.at[pl.ds(row0_of(g), CH)],
                        osems[slot])
            return carry

        lax.fori_loop(0, nch_w, chunk_body, jnp.int32(0))

        for slot in (0, 1):
            @pl.when((nch_w > slot) & (((nch_w - 1 - slot) % 2) == 0))
            def _():
                pltpu.make_async_copy(
                    ob0, out_hbm.at[pl.ds(0, CH)], so0).wait()

            @pl.when((nch_w > slot) & (((nch_w - 1 - slot) % 2) == 1))
            def _():
                pltpu.make_async_copy(
                    ob1, out_hbm.at[pl.ds(0, CH)], so1).wait()

        if tail:
            @pl.when(wid == 0)
            def _():
                row0 = full * CH
                pltpu.sync_copy(exe_hbm.at[pl.ds(row0, tail)],
                                ib0.at[pl.ds(0, tail)])
                compute_rows(ib0, ob0, row0, tail)
                pltpu.sync_copy(ob0.at[pl.ds(0, tail)],
                                out_hbm.at[pl.ds(row0, tail)])

    return sc_exe


def kernel(uid, exe_path, timestamp, edge_type):
    n, d = exe_path.shape
    S, L = 8, 1024
    blk = S * L
    G = -(-n // blk)

    blk1 = pl.BlockSpec((blk,), lambda b: (b,))
    stage_a = pl.pallas_call(
        functools.partial(_attrs_kernel, S, L),
        grid=(G,),
        in_specs=[blk1, blk1, blk1],
        out_specs=[blk1] * 7,
        out_shape=[
            jax.ShapeDtypeStruct((n,), uid.dtype),
            jax.ShapeDtypeStruct((n,), timestamp.dtype),
            jax.ShapeDtypeStruct((n,), edge_type.dtype),
            jax.ShapeDtypeStruct((n,), jnp.bool_),
            jax.ShapeDtypeStruct((n,), jnp.bool_),
            jax.ShapeDtypeStruct((n,), jnp.bool_),
            jax.ShapeDtypeStruct((n,), jnp.bool_),
        ],
        compiler_params=pltpu.CompilerParams(
            dimension_semantics=("arbitrary",)),
    )
    (muid, mts, met, m_uid, m_exe, m_ts, m_et) = stage_a(
        uid, timestamp, edge_type)

    mexe = _make_sc_exe(n, d, 32)(exe_path)

    return muid, mexe, mts, met, m_uid, m_exe, m_ts, m_et
